# baseline (speedup 1.0000x reference)
import jax
import jax.numpy as jnp
from jax import lax
from jax.experimental import pallas as pl
from jax.experimental.pallas import tpu as pltpu

B, D, O, E, H = 128, 768, 768, 8, 7680
HT = 1920
NHT = H // HT
RB = 32
NRB = B // RB


def _gate_body(x1_ref, Wg_ref, bg_ref, gate_ref, cnt_ref):
    cols = lax.broadcasted_iota(jnp.int32, (B, E), 1)
    logits = jnp.dot(x1_ref[...], Wg_ref[...],
                     preferred_element_type=jnp.float32) + bg_ref[...]
    m = jnp.max(logits, axis=1, keepdims=True)
    ex = jnp.exp(logits - m)
    probs = ex / jnp.sum(ex, axis=1, keepdims=True)
    m1 = jnp.max(probs, axis=1, keepdims=True)
    i1 = jnp.min(jnp.where(probs == m1, cols, E), axis=1, keepdims=True)
    mask1 = cols == i1
    probs_wo1 = jnp.where(mask1, -1.0, probs)
    m2 = jnp.max(probs_wo1, axis=1, keepdims=True)
    i2 = jnp.min(jnp.where(probs_wo1 == m2, cols, E), axis=1, keepdims=True)
    routed = jnp.where(mask1 | (cols == i2), probs, 0.0)
    gate_ref[...] = routed
    cnt_ref[...] = jnp.sum((routed > 0.0).astype(jnp.int32), axis=0,
                           keepdims=True)


def _moe_body(cnt_ref, x2_ref, gate_ref, W1_ref, b1_ref, W2_ref, b2_ref,
              out_ref, S_ref, xg_ref, y_ref):
    e = pl.program_id(0)
    ht = pl.program_id(1)
    cnt = cnt_ref[e]
    cols = lax.broadcasted_iota(jnp.int32, (B, E), 1)
    gate = gate_ref[...]
    sel = cols == e
    gcol = jnp.sum(jnp.where(sel, gate, 0.0), axis=1, keepdims=True)

    @pl.when((e == 0) & (ht == 0))
    def _init_out():
        out_ref[...] = jnp.dot(gate, b2_ref[...],
                               preferred_element_type=jnp.float32)

    @pl.when(ht == 0)
    def _gather():
        y_ref[...] = jnp.zeros_like(y_ref)
        mcol = (gcol > 0.0).astype(jnp.float32)
        tri = lax.broadcasted_iota(jnp.int32, (B, B), 1) < \
            lax.broadcasted_iota(jnp.int32, (B, B), 0)
        excl = jnp.where(tri, 1.0, 0.0)
        pos = jnp.dot(excl.astype(jnp.bfloat16), mcol.astype(jnp.bfloat16),
                      preferred_element_type=jnp.float32)
        slot = lax.broadcasted_iota(jnp.int32, (B, B), 1)
        tok_pos = pos.astype(jnp.int32)
        St = jnp.where((slot == tok_pos) & (mcol > 0.0), 1.0, 0.0)
        S_ref[...] = St
        xg_ref[...] = lax.dot_general(
            St.astype(jnp.bfloat16), x2_ref[...].astype(jnp.bfloat16),
            (((0,), (0,)), ((), ())),
            preferred_element_type=jnp.float32)

    for r in range(NRB):
        @pl.when(r * RB < cnt)
        def _block(r=r):
            rows = pl.ds(r * RB, RB)
            hblk = jnp.dot(xg_ref[rows, :].astype(jnp.bfloat16),
                           W1_ref[0].astype(jnp.bfloat16),
                           preferred_element_type=jnp.float32)
            hblk = jnp.maximum(hblk + b1_ref[0], 0.0).astype(jnp.bfloat16)
            yblk = jnp.dot(hblk, W2_ref[0].astype(jnp.bfloat16),
                           preferred_element_type=jnp.float32)

            @pl.when(ht == 0)
            def _set():
                y_ref[rows, :] = yblk

            @pl.when(ht != 0)
            def _acc():
                y_ref[rows, :] += yblk

    @pl.when(ht == NHT - 1)
    def _scatter():
        out_ref[...] += gcol * jnp.dot(
            S_ref[...], y_ref[...],
            preferred_element_type=jnp.float32,
            precision=lax.Precision.HIGHEST)


def kernel(x1, x2, Wg, bg, W1, b1, W2, b2):
    bg2 = bg.reshape(1, E)
    b1_3d = b1.reshape(E, 1, H)
    gate, cnt = pl.pallas_call(
        _gate_body,
        out_shape=(jax.ShapeDtypeStruct((B, E), jnp.float32),
                   jax.ShapeDtypeStruct((1, E), jnp.int32)),
    )(x1, Wg, bg2)
    cnt = cnt.reshape(E)
    grid_spec = pltpu.PrefetchScalarGridSpec(
        num_scalar_prefetch=1,
        grid=(E, NHT),
        in_specs=[
            pl.BlockSpec((B, D), lambda e, h, *_: (0, 0)),
            pl.BlockSpec((B, E), lambda e, h, *_: (0, 0)),
            pl.BlockSpec((1, D, HT), lambda e, h, *_: (e, 0, h)),
            pl.BlockSpec((1, 1, HT), lambda e, h, *_: (e, 0, h)),
            pl.BlockSpec((1, HT, O), lambda e, h, *_: (e, h, 0)),
            pl.BlockSpec((E, O), lambda e, h, *_: (0, 0)),
        ],
        out_specs=pl.BlockSpec((B, O), lambda e, h, *_: (0, 0)),
        scratch_shapes=[
            pltpu.VMEM((B, B), jnp.float32),
            pltpu.VMEM((B, D), jnp.float32),
            pltpu.VMEM((B, O), jnp.float32),
        ],
    )
    return pl.pallas_call(
        _moe_body,
        grid_spec=grid_spec,
        out_shape=jax.ShapeDtypeStruct((B, O), jnp.float32),
        compiler_params=pltpu.CompilerParams(
            dimension_semantics=("arbitrary", "arbitrary"),
        ),
    )(cnt, x2, gate, W1, b1_3d, W2, b2)


# speedup vs baseline: 1.0287x; 1.0287x over previous
import jax
import jax.numpy as jnp
from jax import lax
from jax.experimental import pallas as pl
from jax.experimental.pallas import tpu as pltpu

B, D, O, E, H = 128, 768, 768, 8, 7680
HT = 1920
NHT = H // HT
RB = 64
NRB = B // RB


def _gate_body(x1_ref, Wg_ref, bg_ref, gate_ref, cnt_ref):
    cols = lax.broadcasted_iota(jnp.int32, (B, E), 1)
    logits = jnp.dot(x1_ref[...], Wg_ref[...],
                     preferred_element_type=jnp.float32) + bg_ref[...]
    m = jnp.max(logits, axis=1, keepdims=True)
    ex = jnp.exp(logits - m)
    probs = ex / jnp.sum(ex, axis=1, keepdims=True)
    m1 = jnp.max(probs, axis=1, keepdims=True)
    i1 = jnp.min(jnp.where(probs == m1, cols, E), axis=1, keepdims=True)
    mask1 = cols == i1
    probs_wo1 = jnp.where(mask1, -1.0, probs)
    m2 = jnp.max(probs_wo1, axis=1, keepdims=True)
    i2 = jnp.min(jnp.where(probs_wo1 == m2, cols, E), axis=1, keepdims=True)
    routed = jnp.where(mask1 | (cols == i2), probs, 0.0)
    gate_ref[...] = routed
    cnt_ref[...] = jnp.sum((routed > 0.0).astype(jnp.int32), axis=0,
                           keepdims=True)


def _moe_body(cnt_ref, x2_ref, gate_ref, W1_ref, b1_ref, W2_ref, b2_ref,
              out_ref, S_ref, xg_ref, y_ref):
    e = pl.program_id(0)
    ht = pl.program_id(1)
    cnt = cnt_ref[e]
    cols = lax.broadcasted_iota(jnp.int32, (B, E), 1)
    gate = gate_ref[...]
    sel = cols == e
    gcol = jnp.sum(jnp.where(sel, gate, 0.0), axis=1, keepdims=True)

    @pl.when((e == 0) & (ht == 0))
    def _init_out():
        out_ref[...] = jnp.dot(gate, b2_ref[...],
                               preferred_element_type=jnp.float32)

    @pl.when(ht == 0)
    def _gather():
        y_ref[...] = jnp.zeros_like(y_ref)
        mcol = (gcol > 0.0).astype(jnp.float32)
        tri = lax.broadcasted_iota(jnp.int32, (B, B), 1) < \
            lax.broadcasted_iota(jnp.int32, (B, B), 0)
        excl = jnp.where(tri, 1.0, 0.0)
        pos = jnp.dot(excl.astype(jnp.bfloat16), mcol.astype(jnp.bfloat16),
                      preferred_element_type=jnp.float32)
        slot = lax.broadcasted_iota(jnp.int32, (B, B), 1)
        tok_pos = pos.astype(jnp.int32)
        St = jnp.where((slot == tok_pos) & (mcol > 0.0), 1.0, 0.0)
        S_ref[...] = St
        xg_ref[...] = lax.dot_general(
            St.astype(jnp.bfloat16), x2_ref[...].astype(jnp.bfloat16),
            (((0,), (0,)), ((), ())),
            preferred_element_type=jnp.float32)

    for r in range(NRB):
        @pl.when(r * RB < cnt)
        def _block(r=r):
            rows = pl.ds(r * RB, RB)
            hblk = jnp.dot(xg_ref[rows, :].astype(jnp.bfloat16),
                           W1_ref[0].astype(jnp.bfloat16),
                           preferred_element_type=jnp.float32)
            hblk = jnp.maximum(hblk + b1_ref[0], 0.0).astype(jnp.bfloat16)
            yblk = jnp.dot(hblk, W2_ref[0].astype(jnp.bfloat16),
                           preferred_element_type=jnp.float32)

            @pl.when(ht == 0)
            def _set():
                y_ref[rows, :] = yblk

            @pl.when(ht != 0)
            def _acc():
                y_ref[rows, :] += yblk

    @pl.when(ht == NHT - 1)
    def _scatter():
        out_ref[...] += gcol * jnp.dot(
            S_ref[...], y_ref[...],
            preferred_element_type=jnp.float32,
            precision=lax.Precision.HIGHEST)


def kernel(x1, x2, Wg, bg, W1, b1, W2, b2):
    bg2 = bg.reshape(1, E)
    b1_3d = b1.reshape(E, 1, H)
    gate, cnt = pl.pallas_call(
        _gate_body,
        out_shape=(jax.ShapeDtypeStruct((B, E), jnp.float32),
                   jax.ShapeDtypeStruct((1, E), jnp.int32)),
    )(x1, Wg, bg2)
    cnt = cnt.reshape(E)
    grid_spec = pltpu.PrefetchScalarGridSpec(
        num_scalar_prefetch=1,
        grid=(E, NHT),
        in_specs=[
            pl.BlockSpec((B, D), lambda e, h, *_: (0, 0)),
            pl.BlockSpec((B, E), lambda e, h, *_: (0, 0)),
            pl.BlockSpec((1, D, HT), lambda e, h, *_: (e, 0, h)),
            pl.BlockSpec((1, 1, HT), lambda e, h, *_: (e, 0, h)),
            pl.BlockSpec((1, HT, O), lambda e, h, *_: (e, h, 0)),
            pl.BlockSpec((E, O), lambda e, h, *_: (0, 0)),
        ],
        out_specs=pl.BlockSpec((B, O), lambda e, h, *_: (0, 0)),
        scratch_shapes=[
            pltpu.VMEM((B, B), jnp.float32),
            pltpu.VMEM((B, D), jnp.float32),
            pltpu.VMEM((B, O), jnp.float32),
        ],
    )
    return pl.pallas_call(
        _moe_body,
        grid_spec=grid_spec,
        out_shape=jax.ShapeDtypeStruct((B, O), jnp.float32),
        compiler_params=pltpu.CompilerParams(
            dimension_semantics=("arbitrary", "arbitrary"),
        ),
    )(cnt, x2, gate, W1, b1_3d, W2, b2)
